# baseline (speedup 1.0000x reference)
import functools

import jax
import jax.numpy as jnp
from jax import lax
from jax.experimental import pallas as pl
from jax.experimental.pallas import tpu as pltpu
from jax.experimental.pallas import tpu_sc as plsc


def _chol_kernel(lr_ref, diag_ref, y_ref, e_ref, tril_ref, idx_ref, *, e_size):
    idx_ref[...] = y_ref[...] * e_size + e_ref[...]

    lr = lr_ref[...]
    d = diag_ref[...]
    k_comb, z, _ = lr.shape

    cov = lax.dot_general(
        lr, lr,
        dimension_numbers=(((2,), (2,)), ((0,), (0,))),
        preferred_element_type=jnp.float32,
    )
    sp = jnp.maximum(d, 0.0) + jnp.log(1.0 + jnp.exp(-jnp.abs(d)))
    rows = lax.broadcasted_iota(jnp.int32, (k_comb, z, z), 1)
    cols = lax.broadcasted_iota(jnp.int32, (k_comb, z, z), 2)
    a = cov + jnp.where(rows == cols, sp[:, :, None], 0.0)

    def body(k, a):
        dk = jnp.sum(jnp.where((rows == k) & (cols == k), a, 0.0),
                     axis=(1, 2), keepdims=True)
        inv_piv = lax.rsqrt(dk)
        piv = dk * inv_piv
        colk = jnp.sum(jnp.where(cols == k, a, 0.0),
                       axis=2, keepdims=True)
        ridx = rows[:, :, :1]
        v = jnp.where(ridx > k, colk * inv_piv, 0.0)
        a = a - v * jnp.swapaxes(v, 1, 2)
        newcol = jnp.where(ridx == k, piv, v)
        return jnp.where(cols == k, newcol, a)

    a = lax.fori_loop(0, z, body, a)
    tril_ref[...] = jnp.where(rows >= cols, a, 0.0)


def _make_sc_gather(b, z_pad, d_tril, k_comb, nc, ns):
    nw = nc * ns
    b_per_w = b // nw
    loc_chunk = 128
    chunk = 8
    nbuf = 2
    n_chunks = b_per_w // chunk

    mesh = plsc.VectorSubcoreMesh(core_axis_name="c", subcore_axis_name="s")

    @functools.partial(
        pl.kernel,
        out_type=(
            jax.ShapeDtypeStruct((b, z_pad), jnp.float32),
            jax.ShapeDtypeStruct((b, d_tril), jnp.float32),
        ),
        mesh=mesh,
        scratch_types=[
            pltpu.VMEM((b_per_w,), jnp.int32),
            pltpu.VMEM((loc_chunk, z_pad), jnp.float32),
            [pltpu.VMEM((chunk, d_tril), jnp.float32)] * nbuf,
            [pltpu.SemaphoreType.DMA] * nbuf,
            [pltpu.SemaphoreType.DMA] * nbuf,
            pltpu.SemaphoreType.DMA,
        ],
    )
    def sc_gather(mu_hbm, table_hbm, idx_hbm, loc_out, tril_out,
                  idx_v, loc_buf, bufs, gsems, wsems, sem):
        wid = lax.axis_index("s") * nc + lax.axis_index("c")
        base = wid * b_per_w
        pltpu.sync_copy(idx_hbm.at[pl.ds(base, b_per_w)], idx_v)

        for g in range(b_per_w // loc_chunk):
            pltpu.async_copy(
                mu_hbm.at[idx_v.at[pl.ds(g * loc_chunk, loc_chunk)]],
                loc_buf, sem).wait()
            pltpu.sync_copy(loc_buf,
                            loc_out.at[pl.ds(base + g * loc_chunk, loc_chunk)])

        def gather_chunk(k, bslot, gsem):
            return pltpu.make_async_copy(
                table_hbm.at[idx_v.at[pl.ds(k * chunk, chunk)]],
                bslot, gsem)

        for bi in range(nbuf):
            gather_chunk(bi, bufs[bi], gsems[bi]).start()

        def body(g, _):
            for bi in range(nbuf):
                k = g * nbuf + bi
                gather_chunk(k, bufs[bi], gsems[bi]).wait()
                pltpu.async_copy(
                    bufs[bi], tril_out.at[pl.ds(base + k * chunk, chunk)],
                    wsems[bi]).wait()
                @pl.when(k + nbuf < n_chunks)
                def _():
                    gather_chunk(k + nbuf, bufs[bi], gsems[bi]).start()
            return 0

        lax.fori_loop(0, n_chunks // nbuf, body, 0)

    return sc_gather


def kernel(mu, low_rank, diag, y, e):
    n_classes, e_size, z = mu.shape
    rank = low_rank.shape[-1]
    k_comb = n_classes * e_size
    b = y.shape[0]

    lr_t = low_rank.reshape(k_comb, z, rank)
    d_t = diag.reshape(k_comb, z)
    mu_t = mu.reshape(k_comb, z)
    y32 = y.astype(jnp.int32)
    e32 = e.astype(jnp.int32)

    tril_t, idx = pl.pallas_call(
        functools.partial(_chol_kernel, e_size=e_size),
        out_shape=(
            jax.ShapeDtypeStruct((k_comb, z, z), jnp.float32),
            jax.ShapeDtypeStruct((b,), jnp.int32),
        ),
    )(lr_t, d_t, y32, e32)

    z_pad = 128
    mu_pad = jnp.pad(mu_t, ((0, 0), (0, z_pad - z)))

    info = plsc.get_sparse_core_info()
    sc_gather = _make_sc_gather(b, z_pad, z * z, k_comb,
                                info.num_cores, info.num_subcores)
    loc_pad, tril_rows = sc_gather(mu_pad, tril_t.reshape(k_comb, z * z), idx)
    return (loc_pad[:, :z], tril_rows.reshape(b, z, z))


# speedup vs baseline: 1.6368x; 1.6368x over previous
import functools

import jax
import jax.numpy as jnp
from jax import lax
from jax.experimental import pallas as pl
from jax.experimental.pallas import tpu as pltpu
from jax.experimental.pallas import tpu_sc as plsc


def _chol_kernel(lr_ref, diag_ref, y_ref, e_ref, tril_ref, idx_ref, *, e_size):
    idx_ref[...] = y_ref[...] * e_size + e_ref[...]

    lr = lr_ref[...]
    d = diag_ref[...]
    k_comb, z, _ = lr.shape

    cov = lax.dot_general(
        lr, lr,
        dimension_numbers=(((2,), (2,)), ((0,), (0,))),
        preferred_element_type=jnp.float32,
    )
    sp = jnp.maximum(d, 0.0) + jnp.log(1.0 + jnp.exp(-jnp.abs(d)))
    rows = lax.broadcasted_iota(jnp.int32, (k_comb, z, z), 1)
    cols = lax.broadcasted_iota(jnp.int32, (k_comb, z, z), 2)
    a = cov + jnp.where(rows == cols, sp[:, :, None], 0.0)

    def body(k, a):
        dk = jnp.sum(jnp.where((rows == k) & (cols == k), a, 0.0),
                     axis=(1, 2), keepdims=True)
        inv_piv = lax.rsqrt(dk)
        piv = dk * inv_piv
        colk = jnp.sum(jnp.where(cols == k, a, 0.0),
                       axis=2, keepdims=True)
        ridx = rows[:, :, :1]
        v = jnp.where(ridx > k, colk * inv_piv, 0.0)
        a = a - v * jnp.swapaxes(v, 1, 2)
        newcol = jnp.where(ridx == k, piv, v)
        return jnp.where(cols == k, newcol, a)

    a = lax.fori_loop(0, z, body, a)
    tril_ref[...] = jnp.where(rows >= cols, a, 0.0)


def _make_sc_gather(b, z_pad, d_tril, k_comb, nc, ns):
    nw = nc * ns
    b_per_w = b // nw
    loc_chunk = 128
    group = 16
    n_groups = b_per_w // group

    mesh = plsc.VectorSubcoreMesh(core_axis_name="c", subcore_axis_name="s")

    @functools.partial(
        pl.kernel,
        out_type=(
            jax.ShapeDtypeStruct((b, z_pad), jnp.float32),
            jax.ShapeDtypeStruct((b, d_tril), jnp.float32),
        ),
        mesh=mesh,
        scratch_types=[
            pltpu.VMEM((b_per_w,), jnp.int32),
            pltpu.VMEM((loc_chunk, z_pad), jnp.float32),
            pltpu.VMEM((k_comb, d_tril), jnp.float32),
            pltpu.VMEM_SHARED((k_comb, d_tril), jnp.float32),
            pltpu.SemaphoreType.DMA,
        ],
    )
    def sc_gather(mu_hbm, table_hbm, idx_hbm, loc_out, tril_out,
                  idx_v, loc_buf, table_v, table_sh, sem):
        cid = lax.axis_index("c")
        sid = lax.axis_index("s")
        wid = sid * nc + cid
        base = wid * b_per_w
        pltpu.sync_copy(idx_hbm.at[pl.ds(base, b_per_w)], idx_v)

        @pl.when(sid == 0)
        def _():
            pltpu.sync_copy(table_hbm, table_v)
            pltpu.sync_copy(table_v, table_sh)
        plsc.subcore_barrier()

        for g in range(b_per_w // loc_chunk):
            pltpu.async_copy(
                mu_hbm.at[idx_v.at[pl.ds(g * loc_chunk, loc_chunk)]],
                loc_buf, sem).wait()
            pltpu.sync_copy(loc_buf,
                            loc_out.at[pl.ds(base + g * loc_chunk, loc_chunk)])

        def body(g, _):
            ivec = idx_v[pl.ds(g * group, group)]
            for j in range(group):
                pltpu.async_copy(
                    table_sh.at[pl.ds(ivec[j], 1)],
                    tril_out.at[pl.ds(base + g * group + j, 1)], sem)
            return 0

        lax.fori_loop(0, n_groups, body, 0)

        def drain(g, _):
            for _j in range(group):
                pltpu.make_async_copy(
                    table_sh.at[pl.ds(0, 1)],
                    tril_out.at[pl.ds(base, 1)], sem).wait()
            return 0

        lax.fori_loop(0, n_groups, drain, 0)

    return sc_gather


def kernel(mu, low_rank, diag, y, e):
    n_classes, e_size, z = mu.shape
    rank = low_rank.shape[-1]
    k_comb = n_classes * e_size
    b = y.shape[0]

    lr_t = low_rank.reshape(k_comb, z, rank)
    d_t = diag.reshape(k_comb, z)
    mu_t = mu.reshape(k_comb, z)
    y32 = y.astype(jnp.int32)
    e32 = e.astype(jnp.int32)

    tril_t, idx = pl.pallas_call(
        functools.partial(_chol_kernel, e_size=e_size),
        out_shape=(
            jax.ShapeDtypeStruct((k_comb, z, z), jnp.float32),
            jax.ShapeDtypeStruct((b,), jnp.int32),
        ),
    )(lr_t, d_t, y32, e32)

    z_pad = 128
    mu_pad = jnp.pad(mu_t, ((0, 0), (0, z_pad - z)))

    info = plsc.get_sparse_core_info()
    sc_gather = _make_sc_gather(b, z_pad, z * z, k_comb,
                                info.num_cores, info.num_subcores)
    loc_pad, tril_rows = sc_gather(mu_pad, tril_t.reshape(k_comb, z * z), idx)
    return (loc_pad[:, :z], tril_rows.reshape(b, z, z))
